# baseline (speedup 1.0000x reference)
import functools

import jax
import jax.numpy as jnp
from jax import lax
from jax.experimental import pallas as pl
from jax.experimental.pallas import tpu as pltpu
from jax.experimental.pallas import tpu_sc as plsc

_N = 10000
_E = 320000
_D = 128

_NC = 2
_NS = 16
_NW = _NC * _NS
_C = 256
_NPH = 2
_HALF = 20
_NCHUNK = _NPH * _HALF
_EPT = _NCHUNK * _C
_EPAD = _NW * _EPT
_NPAD = 8
_NA = _N + _NPAD
_RPT = 632
_RPT_LAST = _N - (_NS - 1) * _RPT
_ZPT_LAST = _NA - (_NS - 1) * _RPT


@functools.lru_cache(maxsize=None)
def _make_seg_sum(with_counts: bool):
    mesh = plsc.VectorSubcoreMesh(core_axis_name="c", subcore_axis_name="s",
                                  num_cores=_NC, num_subcores=_NS)
    out_type = [jax.ShapeDtypeStruct((_NC, _N, _D), jnp.float32)]
    scratch = [
        pltpu.VMEM_SHARED((_NA, _D), jnp.float32),
        pltpu.VMEM((_HALF * _C,), jnp.int32),
        pltpu.VMEM((_HALF * _C,), jnp.int32),
        pltpu.VMEM((_C, _D), jnp.float32),
        pltpu.SemaphoreType.DMA,
    ]
    if with_counts:
        out_type.append(jax.ShapeDtypeStruct((_NC, _NA), jnp.float32))
        scratch.append(pltpu.VMEM_SHARED((_NA,), jnp.float32))
        scratch.append(pltpu.VMEM((_C,), jnp.float32))

    @functools.partial(
        pl.kernel,
        out_type=tuple(out_type),
        mesh=mesh,
        scratch_types=tuple(scratch),
    )
    def seg_sum(x_hbm, src_hbm, dst_hbm, zrows_hbm, zn_hbm, *rest):
        if with_counts:
            sum_out, cnt_out, acc, src_v, dst_v, rows_v, gsem, cnt, ones_v = rest
        else:
            sum_out, acc, src_v, dst_v, rows_v, gsem = rest
        c = lax.axis_index("c")
        s = lax.axis_index("s")
        w = s * _NC + c

        base = pl.multiple_of(s * _RPT, 8)

        @pl.when(s < _NS - 1)
        def _():
            pltpu.sync_copy(zrows_hbm.at[pl.ds(0, _RPT)], acc.at[pl.ds(base, _RPT)])

        @pl.when(s == _NS - 1)
        def _():
            pltpu.sync_copy(zrows_hbm.at[pl.ds(0, _ZPT_LAST)],
                            acc.at[pl.ds(base, _ZPT_LAST)])
        if with_counts:

            @pl.when(s == 0)
            def _():
                pltpu.sync_copy(zn_hbm, cnt)

            for k in range(_C // 16):
                ones_v[pl.ds(k * 16, 16)] = jnp.ones((16,), jnp.float32)

        plsc.subcore_barrier()

        for phase in range(_NPH):
            pltpu.sync_copy(src_hbm.at[w, phase], src_v)
            pltpu.sync_copy(dst_hbm.at[w, phase], dst_v)

            def chunk_body(j, carry):
                off = pl.multiple_of(j * _C, 8)
                sj = src_v.at[pl.ds(off, _C)]
                dj = dst_v.at[pl.ds(off, _C)]
                pltpu.async_copy(x_hbm.at[sj], rows_v, gsem).wait()
                pltpu.sync_copy(rows_v, acc.at[dj], add=True)
                if with_counts:
                    pltpu.sync_copy(ones_v, cnt.at[dj], add=True)
                return carry

            lax.fori_loop(0, _HALF, chunk_body, 0)
        plsc.subcore_barrier()

        @pl.when(s < _NS - 1)
        def _():
            pltpu.sync_copy(acc.at[pl.ds(base, _RPT)],
                            sum_out.at[c, pl.ds(base, _RPT)])

        @pl.when(s == _NS - 1)
        def _():
            pltpu.sync_copy(acc.at[pl.ds(base, _RPT_LAST)],
                            sum_out.at[c, pl.ds(base, _RPT_LAST)])
        if with_counts:

            @pl.when(s == 0)
            def _():
                pltpu.sync_copy(cnt, cnt_out.at[c])

    return seg_sum


def _dense1_body(parts, cnts, x, w_l_t, b_l, w_r_t, gamma, beta, h_ref, cinv_ref):
    cnt = cnts[0] + cnts[1]
    cinv = 1.0 / jnp.maximum(cnt, 1.0)
    agg = (parts[0] + parts[1]) * cinv
    h = (jnp.dot(agg, w_l_t[...], preferred_element_type=jnp.float32)
         + b_l[...]
         + jnp.dot(x[...], w_r_t[...], preferred_element_type=jnp.float32))
    m = jnp.mean(h, axis=0, keepdims=True)
    v = jnp.mean((h - m) * (h - m), axis=0, keepdims=True)
    h = (h - m) * lax.rsqrt(v + 1e-5) * gamma[...] + beta[...]
    h_ref[...] = jnp.where(h >= 0, h, 0.01 * h)
    cinv_ref[...] = cinv


def _dense2_body(parts, cinv, h, w_l_t, b_l, w_r_t, out_ref):
    agg = (parts[0] + parts[1]) * cinv[...]
    out_ref[...] = (jnp.dot(agg, w_l_t[...], preferred_element_type=jnp.float32)
                    + b_l[...]
                    + jnp.dot(h[...], w_r_t[...], preferred_element_type=jnp.float32))


_dense1 = pl.pallas_call(
    _dense1_body,
    out_shape=(jax.ShapeDtypeStruct((_N, _D), jnp.float32),
               jax.ShapeDtypeStruct((_N, 1), jnp.float32)),
)

_dense2 = pl.pallas_call(
    _dense2_body,
    out_shape=jax.ShapeDtypeStruct((_N, _D), jnp.float32),
)


def kernel(x, edge_index, W1_l, b1_l, W1_r, bn_gamma, bn_beta, W2_l, b2_l, W2_r):
    pad = _EPAD - _E
    src = jnp.concatenate(
        [edge_index[0], jnp.zeros((pad,), jnp.int32)]
    ).reshape(_NW, _NPH, _HALF * _C)
    dst = jnp.concatenate(
        [edge_index[1], jnp.full((pad,), _N, jnp.int32)]
    ).reshape(_NW, _NPH, _HALF * _C)
    zrows = jnp.zeros((_RPT, _D), jnp.float32)
    zn = jnp.zeros((_NA,), jnp.float32)

    parts1, cnts = _make_seg_sum(True)(x, src, dst, zrows, zn)
    h, cinv = _dense1(parts1, cnts[:, :_N, None], x, W1_l.T, b1_l[None, :],
                      W1_r.T, bn_gamma[None, :], bn_beta[None, :])
    (parts2,) = _make_seg_sum(False)(h, src, dst, zrows, zn)
    out = _dense2(parts2, cinv, h, W2_l.T, b2_l[None, :], W2_r.T)
    return out


# speedup vs baseline: 2.5206x; 2.5206x over previous
import functools

import jax
import jax.numpy as jnp
from jax import lax
from jax.experimental import pallas as pl
from jax.experimental.pallas import tpu as pltpu
from jax.experimental.pallas import tpu_sc as plsc

_N = 10000
_E = 320000
_D = 128

_NC = 2
_NS = 16
_NW = _NC * _NS
_EPW = _E // _NW
_C = 125
_NCHUNK = _EPW // _C
_RPT = 632
_RPT_LAST = _N - (_NS - 1) * _RPT


@functools.lru_cache(maxsize=None)
def _make_seg_sum(with_counts: bool):
    mesh = plsc.VectorSubcoreMesh(core_axis_name="c", subcore_axis_name="s",
                                  num_cores=_NC, num_subcores=_NS)
    out_type = [jax.ShapeDtypeStruct((_NC, _N, _D), jnp.float32)]
    scratch = [
        pltpu.VMEM_SHARED((_N, _D), jnp.float32),
        pltpu.VMEM((_NCHUNK, _C), jnp.int32),
        pltpu.VMEM((_NCHUNK, _C), jnp.int32),
        pltpu.VMEM((_C, _D), jnp.float32),
        pltpu.SemaphoreType.DMA,
    ]
    if with_counts:
        out_type.append(jax.ShapeDtypeStruct((_NC, _N), jnp.float32))
        scratch.append(pltpu.VMEM_SHARED((_N,), jnp.float32))
        scratch.append(pltpu.VMEM((128,), jnp.float32))

    @functools.partial(
        pl.kernel,
        out_type=tuple(out_type),
        mesh=mesh,
        scratch_types=tuple(scratch),
    )
    def seg_sum(x_hbm, src_hbm, dst_hbm, zrows_hbm, zn_hbm, *rest):
        if with_counts:
            sum_out, cnt_out, acc, src_v, dst_v, rows_v, gsem, cnt, ones_v = rest
        else:
            sum_out, acc, src_v, dst_v, rows_v, gsem = rest
        c = lax.axis_index("c")
        s = lax.axis_index("s")
        w = s * _NC + c

        base = pl.multiple_of(s * _RPT, 8)

        @pl.when(s < _NS - 1)
        def _():
            pltpu.sync_copy(zrows_hbm.at[pl.ds(0, _RPT)], acc.at[pl.ds(base, _RPT)])

        @pl.when(s == _NS - 1)
        def _():
            pltpu.sync_copy(zrows_hbm.at[pl.ds(0, _RPT_LAST)],
                            acc.at[pl.ds(base, _RPT_LAST)])
        if with_counts:

            @pl.when(s == 0)
            def _():
                pltpu.sync_copy(zn_hbm, cnt)

            for k in range(8):
                ones_v[pl.ds(k * 16, 16)] = jnp.ones((16,), jnp.float32)

        pltpu.sync_copy(src_hbm.at[w], src_v)
        pltpu.sync_copy(dst_hbm.at[w], dst_v)
        plsc.subcore_barrier()

        def chunk_body(j, carry):
            pltpu.async_copy(x_hbm.at[src_v.at[j]], rows_v, gsem).wait()
            pltpu.sync_copy(rows_v, acc.at[dst_v.at[j]], add=True)
            if with_counts:
                pltpu.sync_copy(ones_v.at[pl.ds(0, _C)], cnt.at[dst_v.at[j]], add=True)
            return carry

        lax.fori_loop(0, _NCHUNK, chunk_body, 0)
        plsc.subcore_barrier()

        @pl.when(s < _NS - 1)
        def _():
            pltpu.sync_copy(acc.at[pl.ds(base, _RPT)],
                            sum_out.at[c, pl.ds(base, _RPT)])

        @pl.when(s == _NS - 1)
        def _():
            pltpu.sync_copy(acc.at[pl.ds(base, _RPT_LAST)],
                            sum_out.at[c, pl.ds(base, _RPT_LAST)])
        if with_counts:

            @pl.when(s == 0)
            def _():
                pltpu.sync_copy(cnt, cnt_out.at[c])

    return seg_sum


def _lin_body(a, w_t, out_ref):
    out_ref[...] = jnp.dot(a[...], w_t[...], preferred_element_type=jnp.float32)


def _dense1_body(parts, cnts, xr, w_l_t, b_l, gamma, beta, h_ref, cinv_ref):
    cnt = cnts[0] + cnts[1]
    cinv = 1.0 / jnp.maximum(cnt, 1.0)
    agg = (parts[0] + parts[1]) * cinv
    h = (jnp.dot(agg, w_l_t[...], preferred_element_type=jnp.float32)
         + b_l[...] + xr[...])
    m = jnp.mean(h, axis=0, keepdims=True)
    v = jnp.mean((h - m) * (h - m), axis=0, keepdims=True)
    h = (h - m) * lax.rsqrt(v + 1e-5) * gamma[...] + beta[...]
    h_ref[...] = jnp.where(h >= 0, h, 0.01 * h)
    cinv_ref[...] = cinv


def _dense2_body(parts, cinv, hr, w_l_t, b_l, out_ref):
    agg = (parts[0] + parts[1]) * cinv[...]
    out_ref[...] = (jnp.dot(agg, w_l_t[...], preferred_element_type=jnp.float32)
                    + b_l[...] + hr[...])


_lin = pl.pallas_call(
    _lin_body,
    out_shape=jax.ShapeDtypeStruct((_N, _D), jnp.float32),
)

_dense1 = pl.pallas_call(
    _dense1_body,
    out_shape=(jax.ShapeDtypeStruct((_N, _D), jnp.float32),
               jax.ShapeDtypeStruct((_N, 1), jnp.float32)),
)

_dense2 = pl.pallas_call(
    _dense2_body,
    out_shape=jax.ShapeDtypeStruct((_N, _D), jnp.float32),
)


def kernel(x, edge_index, W1_l, b1_l, W1_r, bn_gamma, bn_beta, W2_l, b2_l, W2_r):
    src = edge_index[0].reshape(_NW, _NCHUNK, _C)
    dst = edge_index[1].reshape(_NW, _NCHUNK, _C)
    zrows = jnp.zeros((_RPT, _D), jnp.float32)
    zn = jnp.zeros((_N,), jnp.float32)

    xr = _lin(x, W1_r.T)
    parts1, cnts = _make_seg_sum(True)(x, src, dst, zrows, zn)
    h, cinv = _dense1(parts1, cnts[:, :, None], xr, W1_l.T, b1_l[None, :],
                      bn_gamma[None, :], bn_beta[None, :])
    hr = _lin(h, W2_r.T)
    (parts2,) = _make_seg_sum(False)(h, src, dst, zrows, zn)
    out = _dense2(parts2, cinv, hr, W2_l.T, b2_l[None, :])
    return out


# speedup vs baseline: 2.5800x; 1.0236x over previous
import functools

import jax
import jax.numpy as jnp
from jax import lax
from jax.experimental import pallas as pl
from jax.experimental.pallas import tpu as pltpu
from jax.experimental.pallas import tpu_sc as plsc

_N = 10000
_E = 320000
_D = 128

_NC = 2
_NS = 16
_NW = _NC * _NS
_EPW = _E // _NW
_C = 125
_NCHUNK = _EPW // _C
_RPT = 632
_RPT_LAST = _N - (_NS - 1) * _RPT


@functools.lru_cache(maxsize=None)
def _make_seg_sum(with_counts: bool):
    mesh = plsc.VectorSubcoreMesh(core_axis_name="c", subcore_axis_name="s",
                                  num_cores=_NC, num_subcores=_NS)
    out_type = [jax.ShapeDtypeStruct((_NC, _N, _D), jnp.float32)]
    scratch = [
        pltpu.VMEM_SHARED((_N, _D), jnp.float32),
        pltpu.VMEM((_NCHUNK, _C), jnp.int32),
        pltpu.VMEM((_NCHUNK, _C), jnp.int32),
        pltpu.VMEM((_C, _D), jnp.float32),
        pltpu.SemaphoreType.DMA,
    ]
    if with_counts:
        out_type.append(jax.ShapeDtypeStruct((_NC, _N), jnp.float32))
        scratch.append(pltpu.VMEM_SHARED((_N,), jnp.float32))
        scratch.append(pltpu.VMEM((128,), jnp.float32))
        scratch.append(pltpu.SemaphoreType.DMA)

    @functools.partial(
        pl.kernel,
        out_type=tuple(out_type),
        mesh=mesh,
        scratch_types=tuple(scratch),
    )
    def seg_sum(x_hbm, src_hbm, dst_hbm, zrows_hbm, zn_hbm, *rest):
        if with_counts:
            (sum_out, cnt_out, acc, src_v, dst_v, rows_v, gsem,
             cnt, ones_v, csem) = rest
        else:
            sum_out, acc, src_v, dst_v, rows_v, gsem = rest
        c = lax.axis_index("c")
        s = lax.axis_index("s")
        w = s * _NC + c

        base = pl.multiple_of(s * _RPT, 8)

        @pl.when(s < _NS - 1)
        def _():
            pltpu.sync_copy(zrows_hbm.at[pl.ds(0, _RPT)], acc.at[pl.ds(base, _RPT)])

        @pl.when(s == _NS - 1)
        def _():
            pltpu.sync_copy(zrows_hbm.at[pl.ds(0, _RPT_LAST)],
                            acc.at[pl.ds(base, _RPT_LAST)])
        if with_counts:

            @pl.when(s == 0)
            def _():
                pltpu.sync_copy(zn_hbm, cnt)

            for k in range(8):
                ones_v[pl.ds(k * 16, 16)] = jnp.ones((16,), jnp.float32)

        pltpu.sync_copy(src_hbm.at[w], src_v)
        pltpu.sync_copy(dst_hbm.at[w], dst_v)
        plsc.subcore_barrier()

        def chunk_body(j, carry):
            pltpu.async_copy(x_hbm.at[src_v.at[j]], rows_v, gsem).wait()
            pltpu.sync_copy(rows_v, acc.at[dst_v.at[j]], add=True)
            if with_counts:
                pltpu.async_copy(ones_v.at[pl.ds(0, _C)], cnt.at[dst_v.at[j]],
                                 csem, add=True)
            return carry

        lax.fori_loop(0, _NCHUNK, chunk_body, 0)
        if with_counts:
            def drain_body(j, carry):
                pltpu.make_async_copy(ones_v.at[pl.ds(0, _C)],
                                      cnt.at[dst_v.at[j]], csem).wait()
                return carry

            lax.fori_loop(0, _NCHUNK, drain_body, 0)
        plsc.subcore_barrier()

        @pl.when(s < _NS - 1)
        def _():
            pltpu.sync_copy(acc.at[pl.ds(base, _RPT)],
                            sum_out.at[c, pl.ds(base, _RPT)])

        @pl.when(s == _NS - 1)
        def _():
            pltpu.sync_copy(acc.at[pl.ds(base, _RPT_LAST)],
                            sum_out.at[c, pl.ds(base, _RPT_LAST)])
        if with_counts:

            @pl.when(s == 0)
            def _():
                pltpu.sync_copy(cnt, cnt_out.at[c])

    return seg_sum


def _dense1_body(parts, cnts, x, w_l_t, b_l, w_r_t, gamma, beta, h_ref, cinv_ref):
    cnt = cnts[0] + cnts[1]
    cinv = 1.0 / jnp.maximum(cnt, 1.0)
    agg = (parts[0] + parts[1]) * cinv
    h = (jnp.dot(agg, w_l_t[...], preferred_element_type=jnp.float32)
         + b_l[...]
         + jnp.dot(x[...], w_r_t[...], preferred_element_type=jnp.float32))
    m = jnp.mean(h, axis=0, keepdims=True)
    v = jnp.mean((h - m) * (h - m), axis=0, keepdims=True)
    h = (h - m) * lax.rsqrt(v + 1e-5) * gamma[...] + beta[...]
    h_ref[...] = jnp.where(h >= 0, h, 0.01 * h)
    cinv_ref[...] = cinv


def _dense2_body(parts, cinv, h, w_l_t, b_l, w_r_t, out_ref):
    agg = (parts[0] + parts[1]) * cinv[...]
    out_ref[...] = (jnp.dot(agg, w_l_t[...], preferred_element_type=jnp.float32)
                    + b_l[...]
                    + jnp.dot(h[...], w_r_t[...], preferred_element_type=jnp.float32))


_dense1 = pl.pallas_call(
    _dense1_body,
    out_shape=(jax.ShapeDtypeStruct((_N, _D), jnp.float32),
               jax.ShapeDtypeStruct((_N, 1), jnp.float32)),
)

_dense2 = pl.pallas_call(
    _dense2_body,
    out_shape=jax.ShapeDtypeStruct((_N, _D), jnp.float32),
)


def kernel(x, edge_index, W1_l, b1_l, W1_r, bn_gamma, bn_beta, W2_l, b2_l, W2_r):
    src = edge_index[0].reshape(_NW, _NCHUNK, _C)
    dst = edge_index[1].reshape(_NW, _NCHUNK, _C)
    zrows = jnp.zeros((_RPT, _D), jnp.float32)
    zn = jnp.zeros((_N,), jnp.float32)

    parts1, cnts = _make_seg_sum(True)(x, src, dst, zrows, zn)
    h, cinv = _dense1(parts1, cnts[:, :, None], x, W1_l.T, b1_l[None, :],
                      W1_r.T, bn_gamma[None, :], bn_beta[None, :])
    (parts2,) = _make_seg_sum(False)(h, src, dst, zrows, zn)
    out = _dense2(parts2, cinv, h, W2_l.T, b2_l[None, :], W2_r.T)
    return out
